# baseline (speedup 1.0000x reference)
import functools

import jax
import jax.numpy as jnp
from jax import lax
from jax.experimental import pallas as pl
from jax.experimental.pallas import tpu as pltpu
from jax.experimental.pallas import tpu_sc as plsc

N = 10000
D = 128
NC = 2
NS = 16
NW = NC * NS
CHUNK = 120
CHUNKS = 84
EPW = CHUNK * CHUNKS
E_PAD = NW * EPW
N_PAD = 10240
RPT = N_PAD // NS
BLK = 1024


def _mesh():
    return plsc.VectorSubcoreMesh(
        core_axis_name="c", subcore_axis_name="s", num_cores=NC, num_subcores=NS
    )


def _deg_body(dst_hbm, out_hbm, idx_v, ones_v, zeros_v, hist_sh):
    c = lax.axis_index("c")
    s = lax.axis_index("s")
    wid = c * NS + s
    pltpu.sync_copy(dst_hbm.at[wid], idx_v)
    for i in range(128 // 16):
        ones_v[pl.ds(i * 16, 16)] = jnp.ones((16,), jnp.float32)
    for i in range(640 // 16):
        zeros_v[pl.ds(i * 16, 16)] = jnp.zeros((16,), jnp.float32)
    pltpu.sync_copy(zeros_v.at[pl.ds(0, RPT)], hist_sh.at[pl.ds(s * RPT, RPT)])
    plsc.subcore_barrier()

    def body(j, carry):
        pltpu.sync_copy(ones_v.at[pl.ds(0, CHUNK)], hist_sh.at[idx_v.at[j, 0]], add=True)
        return carry

    lax.fori_loop(0, CHUNKS, body, 0)
    plsc.subcore_barrier()
    pltpu.sync_copy(
        hist_sh.at[pl.ds(s * RPT, RPT)], out_hbm.at[c, pl.ds(s * RPT, RPT)]
    )


_deg_call = functools.partial(
    pl.kernel,
    out_type=jax.ShapeDtypeStruct((NC, N_PAD), jnp.float32),
    mesh=_mesh(),
    scratch_types=[
        pltpu.VMEM((CHUNKS, 1, CHUNK), jnp.int32),
        pltpu.VMEM((128,), jnp.float32),
        pltpu.VMEM((640,), jnp.float32),
        pltpu.VMEM_SHARED((N_PAD,), jnp.float32),
    ],
)(_deg_body)


NBUF = 3


def _agg_body(y_hbm, src_hbm, dst_hbm, out_hbm, sidx_v, didx_v, buf_v, agg_sh, *sems):
    isem = sems[:NBUF]
    jsem = sems[NBUF : 2 * NBUF]
    gsem = sems[2 * NBUF : 3 * NBUF]
    ssem = sems[3 * NBUF :]
    c = lax.axis_index("c")
    s = lax.axis_index("s")
    wid = c * NS + s
    pltpu.sync_copy(y_hbm.at[pl.ds(s * RPT, RPT)], agg_sh.at[pl.ds(s * RPT, RPT)])
    plsc.subcore_barrier()

    def idx_start(j, b):
        pltpu.async_copy(src_hbm.at[wid, j], sidx_v.at[b], isem[b])
        pltpu.async_copy(dst_hbm.at[wid, j], didx_v.at[b], jsem[b])

    def idx_wait(j, b):
        pltpu.make_async_copy(src_hbm.at[wid, j], sidx_v.at[b], isem[b]).wait()
        pltpu.make_async_copy(dst_hbm.at[wid, j], didx_v.at[b], jsem[b]).wait()

    def gather(b):
        pltpu.async_copy(y_hbm.at[sidx_v.at[b, 0]], buf_v.at[b], gsem[b])

    def gather_wait(b):
        pltpu.make_async_copy(y_hbm.at[sidx_v.at[b, 0]], buf_v.at[b], gsem[b]).wait()

    def scat(b):
        pltpu.async_copy(buf_v.at[b], agg_sh.at[didx_v.at[b, 0]], ssem[b], add=True)

    def scat_wait(b):
        pltpu.make_async_copy(
            buf_v.at[b], agg_sh.at[didx_v.at[b, 0]], ssem[b]
        ).wait()

    for b in range(NBUF):
        idx_start(b, b)

    def body(t, carry):
        j0 = t * NBUF
        for b in range(NBUF):
            idx_wait(j0 + b, b)
            gather(b)
        for b in range(NBUF):
            gather_wait(b)
            scat(b)
        for b in range(NBUF):
            scat_wait(b)
            idx_start(j0 + NBUF + b, b)
        return carry

    lax.fori_loop(0, CHUNKS // NBUF - 1, body, 0)
    j0 = CHUNKS - NBUF
    for b in range(NBUF):
        idx_wait(j0 + b, b)
        gather(b)
    for b in range(NBUF):
        gather_wait(b)
        scat(b)
    for b in range(NBUF):
        scat_wait(b)

    plsc.subcore_barrier()
    pltpu.sync_copy(
        agg_sh.at[pl.ds(s * RPT, RPT)], out_hbm.at[c, pl.ds(s * RPT, RPT)]
    )


_agg_call = functools.partial(
    pl.kernel,
    out_type=jax.ShapeDtypeStruct((NC, N_PAD, D), jnp.float32),
    mesh=_mesh(),
    scratch_types=[
        pltpu.VMEM((NBUF, 1, CHUNK), jnp.int32),
        pltpu.VMEM((NBUF, 1, CHUNK), jnp.int32),
        pltpu.VMEM((NBUF, CHUNK, D), jnp.float32),
        pltpu.VMEM_SHARED((N_PAD, D), jnp.float32),
    ]
    + [pltpu.SemaphoreType.DMA] * (4 * NBUF),
)(_agg_body)


def _scale_mm_body(hist_ref, x_ref, w_ref, dis_ref, y_ref):
    hist = hist_ref[...]
    dis = lax.rsqrt(hist[0] + hist[1] + 1.0)
    dis_ref[...] = dis
    xw = jnp.dot(x_ref[...], w_ref[...], preferred_element_type=jnp.float32)
    y_ref[...] = xw * dis


def _tc_scale_mm(hist3, xp, W1):
    return pl.pallas_call(
        _scale_mm_body,
        grid=(N_PAD // BLK,),
        in_specs=[
            pl.BlockSpec((NC, BLK, 1), lambda i: (0, i, 0)),
            pl.BlockSpec((BLK, D), lambda i: (i, 0)),
            pl.BlockSpec((D, D), lambda i: (0, 0)),
        ],
        out_specs=[
            pl.BlockSpec((BLK, 1), lambda i: (i, 0)),
            pl.BlockSpec((BLK, D), lambda i: (i, 0)),
        ],
        out_shape=[
            jax.ShapeDtypeStruct((N_PAD, 1), jnp.float32),
            jax.ShapeDtypeStruct((N_PAD, D), jnp.float32),
        ],
    )(hist3, xp, W1)


def _mid_body(agg_ref, y1_ref, dis_ref, b1_ref, w2_ref, y2_ref):
    a = agg_ref[...]
    dis = dis_ref[...]
    pre = dis * (a[0] + a[1] - y1_ref[...]) + b1_ref[...]
    h = jnp.maximum(pre, 0.0)
    hw = jnp.dot(h, w2_ref[...], preferred_element_type=jnp.float32)
    y2_ref[...] = hw * dis


def _tc_mid(agg1, y1, dis, b1, W2):
    return pl.pallas_call(
        _mid_body,
        grid=(N_PAD // BLK,),
        in_specs=[
            pl.BlockSpec((NC, BLK, D), lambda i: (0, i, 0)),
            pl.BlockSpec((BLK, D), lambda i: (i, 0)),
            pl.BlockSpec((BLK, 1), lambda i: (i, 0)),
            pl.BlockSpec((D,), lambda i: (0,)),
            pl.BlockSpec((D, D), lambda i: (0, 0)),
        ],
        out_specs=pl.BlockSpec((BLK, D), lambda i: (i, 0)),
        out_shape=jax.ShapeDtypeStruct((N_PAD, D), jnp.float32),
    )(agg1, y1, dis, b1, W2)


def _final_body(agg_ref, y2_ref, dis_ref, b2_ref, out_ref):
    a = agg_ref[...]
    z = dis_ref[...] * (a[0] + a[1] - y2_ref[...]) + b2_ref[...]
    m = jnp.max(z, axis=1, keepdims=True)
    lse = jnp.log(jnp.sum(jnp.exp(z - m), axis=1, keepdims=True))
    out_ref[...] = (z - m) - lse


def _tc_final(agg2, y2, dis, b2):
    return pl.pallas_call(
        _final_body,
        grid=(N_PAD // BLK,),
        in_specs=[
            pl.BlockSpec((NC, BLK, D), lambda i: (0, i, 0)),
            pl.BlockSpec((BLK, D), lambda i: (i, 0)),
            pl.BlockSpec((BLK, 1), lambda i: (i, 0)),
            pl.BlockSpec((D,), lambda i: (0,)),
        ],
        out_specs=pl.BlockSpec((BLK, D), lambda i: (i, 0)),
        out_shape=jax.ShapeDtypeStruct((N_PAD, D), jnp.float32),
    )(agg2, y2, dis, b2)


@jax.jit
def kernel(x, edge_index, W1, b1, W2, b2):
    src = edge_index[0].astype(jnp.int32)
    dst = edge_index[1].astype(jnp.int32)
    e = src.shape[0]
    pad = N + (jnp.arange(E_PAD - e, dtype=jnp.int32) % (N_PAD - N))
    srcp = jnp.concatenate([src, pad]).reshape(NW, CHUNKS, 1, CHUNK)
    dstp = jnp.concatenate([dst, pad]).reshape(NW, CHUNKS, 1, CHUNK)
    xp = jnp.pad(x, ((0, N_PAD - N), (0, 0)))

    hist = _deg_call(dstp)
    hist3 = hist.reshape(NC, N_PAD, 1)
    dis, y1 = _tc_scale_mm(hist3, xp, W1)
    agg1 = _agg_call(y1, srcp, dstp)
    y2 = _tc_mid(agg1, y1, dis, b1, W2)
    agg2 = _agg_call(y2, srcp, dstp)
    outp = _tc_final(agg2, y2, dis, b2)
    return outp[:N]


# speedup vs baseline: 1.0151x; 1.0151x over previous
import functools

import jax
import jax.numpy as jnp
from jax import lax
from jax.experimental import pallas as pl
from jax.experimental.pallas import tpu as pltpu
from jax.experimental.pallas import tpu_sc as plsc

N = 10000
D = 128
NC = 2
NS = 16
NW = NC * NS
CHUNK = 112
CHUNKS = 96
EPW = CHUNK * CHUNKS
E_PAD = NW * EPW
N_PAD = 10240
RPT = N_PAD // NS
BLK = 1024


def _mesh():
    return plsc.VectorSubcoreMesh(
        core_axis_name="c", subcore_axis_name="s", num_cores=NC, num_subcores=NS
    )


def _deg_body(dst_hbm, out_hbm, idx_v, ones_v, zeros_v, hist_sh):
    c = lax.axis_index("c")
    s = lax.axis_index("s")
    wid = c * NS + s
    pltpu.sync_copy(dst_hbm.at[wid], idx_v)
    for i in range(CHUNK // 16):
        ones_v[pl.ds(i * 16, 16)] = jnp.ones((16,), jnp.float32)
    for i in range(RPT // 16):
        zeros_v[pl.ds(i * 16, 16)] = jnp.zeros((16,), jnp.float32)
    pltpu.sync_copy(zeros_v, hist_sh.at[pl.ds(s * RPT, RPT)])
    plsc.subcore_barrier()

    def body(j, carry):
        pltpu.sync_copy(ones_v, hist_sh.at[idx_v.at[j]], add=True)
        return carry

    lax.fori_loop(0, CHUNKS, body, 0)
    plsc.subcore_barrier()
    pltpu.sync_copy(
        hist_sh.at[pl.ds(s * RPT, RPT)], out_hbm.at[c, pl.ds(s * RPT, RPT)]
    )


_deg_call = functools.partial(
    pl.kernel,
    out_type=jax.ShapeDtypeStruct((NC, N_PAD), jnp.float32),
    mesh=_mesh(),
    scratch_types=[
        pltpu.VMEM((CHUNKS, CHUNK), jnp.int32),
        pltpu.VMEM((CHUNK,), jnp.float32),
        pltpu.VMEM((RPT,), jnp.float32),
        pltpu.VMEM_SHARED((N_PAD,), jnp.float32),
    ],
)(_deg_body)


NBUF = 3


PH = 4
CPP = CHUNKS // PH


def _agg_body(y_hbm, src_hbm, dst_hbm, out_hbm, src_v, dst_v, buf_v, agg_sh, *sems):
    gsem = sems[:NBUF]
    ssem = sems[NBUF:]
    c = lax.axis_index("c")
    s = lax.axis_index("s")
    wid = c * NS + s
    pltpu.sync_copy(y_hbm.at[pl.ds(s * RPT, RPT)], agg_sh.at[pl.ds(s * RPT, RPT)])
    plsc.subcore_barrier()

    def gather(j, b):
        pltpu.async_copy(y_hbm.at[src_v.at[j]], buf_v.at[b], gsem[b])

    def gather_wait(j, b):
        pltpu.make_async_copy(y_hbm.at[src_v.at[j]], buf_v.at[b], gsem[b]).wait()

    def scat(j, b):
        pltpu.async_copy(buf_v.at[b], agg_sh.at[dst_v.at[j]], ssem[b], add=True)

    def scat_wait(j, b):
        pltpu.make_async_copy(buf_v.at[b], agg_sh.at[dst_v.at[j]], ssem[b]).wait()

    for p in range(PH):
        pltpu.sync_copy(src_hbm.at[wid, pl.ds(p * CPP, CPP)], src_v)
        pltpu.sync_copy(dst_hbm.at[wid, pl.ds(p * CPP, CPP)], dst_v)

        for b in range(NBUF):
            gather(b, b)

        def body(t, carry):
            j0 = t * NBUF
            for b in range(NBUF):
                gather_wait(j0 + b, b)
                scat(j0 + b, b)
            for b in range(NBUF):
                scat_wait(j0 + b, b)
                gather(j0 + NBUF + b, b)
            return carry

        lax.fori_loop(0, CPP // NBUF - 1, body, 0)
        j0 = CPP - NBUF
        for b in range(NBUF):
            gather_wait(j0 + b, b)
            scat(j0 + b, b)
        for b in range(NBUF):
            scat_wait(j0 + b, b)

    plsc.subcore_barrier()
    pltpu.sync_copy(
        agg_sh.at[pl.ds(s * RPT, RPT)], out_hbm.at[c, pl.ds(s * RPT, RPT)]
    )


_agg_call = functools.partial(
    pl.kernel,
    out_type=jax.ShapeDtypeStruct((NC, N_PAD, D), jnp.float32),
    mesh=_mesh(),
    scratch_types=[
        pltpu.VMEM((CPP, CHUNK), jnp.int32),
        pltpu.VMEM((CPP, CHUNK), jnp.int32),
        pltpu.VMEM((NBUF, CHUNK, D), jnp.float32),
        pltpu.VMEM_SHARED((N_PAD, D), jnp.float32),
    ]
    + [pltpu.SemaphoreType.DMA] * (2 * NBUF),
)(_agg_body)


def _mm_body(x_ref, w_ref, u_ref):
    u_ref[...] = jnp.dot(x_ref[...], w_ref[...], preferred_element_type=jnp.float32)


def _tc_mm(xp, W1):
    return pl.pallas_call(
        _mm_body,
        grid=(N_PAD // BLK,),
        in_specs=[
            pl.BlockSpec((BLK, D), lambda i: (i, 0)),
            pl.BlockSpec((D, D), lambda i: (0, 0)),
        ],
        out_specs=pl.BlockSpec((BLK, D), lambda i: (i, 0)),
        out_shape=jax.ShapeDtypeStruct((N_PAD, D), jnp.float32),
    )(xp, W1)


def _scale_body(hist_ref, u_ref, dis_ref, y_ref):
    hist = hist_ref[...]
    dis = lax.rsqrt(hist[0] + hist[1] + 1.0)
    dis_ref[...] = dis
    y_ref[...] = u_ref[...] * dis


def _tc_scale(hist3, u1):
    return pl.pallas_call(
        _scale_body,
        grid=(N_PAD // BLK,),
        in_specs=[
            pl.BlockSpec((NC, BLK, 1), lambda i: (0, i, 0)),
            pl.BlockSpec((BLK, D), lambda i: (i, 0)),
        ],
        out_specs=[
            pl.BlockSpec((BLK, 1), lambda i: (i, 0)),
            pl.BlockSpec((BLK, D), lambda i: (i, 0)),
        ],
        out_shape=[
            jax.ShapeDtypeStruct((N_PAD, 1), jnp.float32),
            jax.ShapeDtypeStruct((N_PAD, D), jnp.float32),
        ],
    )(hist3, u1)


def _mid_body(agg_ref, y1_ref, dis_ref, b1_ref, w2_ref, y2_ref):
    a = agg_ref[...]
    dis = dis_ref[...]
    pre = dis * (a[0] + a[1] - y1_ref[...]) + b1_ref[...]
    h = jnp.maximum(pre, 0.0)
    hw = jnp.dot(h, w2_ref[...], preferred_element_type=jnp.float32)
    y2_ref[...] = hw * dis


def _tc_mid(agg1, y1, dis, b1, W2):
    return pl.pallas_call(
        _mid_body,
        grid=(N_PAD // BLK,),
        in_specs=[
            pl.BlockSpec((NC, BLK, D), lambda i: (0, i, 0)),
            pl.BlockSpec((BLK, D), lambda i: (i, 0)),
            pl.BlockSpec((BLK, 1), lambda i: (i, 0)),
            pl.BlockSpec((D,), lambda i: (0,)),
            pl.BlockSpec((D, D), lambda i: (0, 0)),
        ],
        out_specs=pl.BlockSpec((BLK, D), lambda i: (i, 0)),
        out_shape=jax.ShapeDtypeStruct((N_PAD, D), jnp.float32),
    )(agg1, y1, dis, b1, W2)


def _final_body(agg_ref, y2_ref, dis_ref, b2_ref, out_ref):
    a = agg_ref[...]
    z = dis_ref[...] * (a[0] + a[1] - y2_ref[...]) + b2_ref[...]
    m = jnp.max(z, axis=1, keepdims=True)
    lse = jnp.log(jnp.sum(jnp.exp(z - m), axis=1, keepdims=True))
    out_ref[...] = (z - m) - lse


def _tc_final(agg2, y2, dis, b2):
    return pl.pallas_call(
        _final_body,
        grid=(N_PAD // BLK,),
        in_specs=[
            pl.BlockSpec((NC, BLK, D), lambda i: (0, i, 0)),
            pl.BlockSpec((BLK, D), lambda i: (i, 0)),
            pl.BlockSpec((BLK, 1), lambda i: (i, 0)),
            pl.BlockSpec((D,), lambda i: (0,)),
        ],
        out_specs=pl.BlockSpec((BLK, D), lambda i: (i, 0)),
        out_shape=jax.ShapeDtypeStruct((N_PAD, D), jnp.float32),
    )(agg2, y2, dis, b2)


@jax.jit
def kernel(x, edge_index, W1, b1, W2, b2):
    src = edge_index[0].astype(jnp.int32)
    dst = edge_index[1].astype(jnp.int32)
    e = src.shape[0]
    pad = N + (jnp.arange(E_PAD - e, dtype=jnp.int32) % (N_PAD - N))
    srcp = jnp.concatenate([src, pad]).reshape(NW, CHUNKS, CHUNK)
    dstp = jnp.concatenate([dst, pad]).reshape(NW, CHUNKS, CHUNK)
    xp = jnp.pad(x, ((0, N_PAD - N), (0, 0)))

    hist = _deg_call(dstp)
    u1 = _tc_mm(xp, W1)
    hist3 = hist.reshape(NC, N_PAD, 1)
    dis, y1 = _tc_scale(hist3, u1)
    agg1 = _agg_call(y1, srcp, dstp)
    y2 = _tc_mid(agg1, y1, dis, b1, W2)
    agg2 = _agg_call(y2, srcp, dstp)
    outp = _tc_final(agg2, y2, dis, b2)
    return outp[:N]


# speedup vs baseline: 1.0213x; 1.0061x over previous
import functools

import jax
import jax.numpy as jnp
from jax import lax
from jax.experimental import pallas as pl
from jax.experimental.pallas import tpu as pltpu
from jax.experimental.pallas import tpu_sc as plsc

N = 10000
D = 128
NC = 2
NS = 16
NW = NC * NS
CHUNK = 112
CHUNKS = 96
EPW = CHUNK * CHUNKS
E_PAD = NW * EPW
N_PAD = 10240
RPT = N_PAD // NS
BLK = 1024


def _mesh():
    return plsc.VectorSubcoreMesh(
        core_axis_name="c", subcore_axis_name="s", num_cores=NC, num_subcores=NS
    )


def _deg_body(dst_hbm, out_hbm, idx_v, ones_v, zeros_v, hist_sh):
    c = lax.axis_index("c")
    s = lax.axis_index("s")
    wid = c * NS + s
    pltpu.sync_copy(dst_hbm.at[wid], idx_v)
    for i in range(CHUNK // 16):
        ones_v[pl.ds(i * 16, 16)] = jnp.ones((16,), jnp.float32)
    for i in range(RPT // 16):
        zeros_v[pl.ds(i * 16, 16)] = jnp.zeros((16,), jnp.float32)
    pltpu.sync_copy(zeros_v, hist_sh.at[pl.ds(s * RPT, RPT)])
    plsc.subcore_barrier()

    def body(j, carry):
        pltpu.sync_copy(ones_v, hist_sh.at[idx_v.at[j]], add=True)
        return carry

    lax.fori_loop(0, CHUNKS, body, 0)
    plsc.subcore_barrier()
    pltpu.sync_copy(
        hist_sh.at[pl.ds(s * RPT, RPT)], out_hbm.at[c, pl.ds(s * RPT, RPT)]
    )


_deg_call = functools.partial(
    pl.kernel,
    out_type=jax.ShapeDtypeStruct((NC, N_PAD), jnp.float32),
    mesh=_mesh(),
    scratch_types=[
        pltpu.VMEM((CHUNKS, CHUNK), jnp.int32),
        pltpu.VMEM((CHUNK,), jnp.float32),
        pltpu.VMEM((RPT,), jnp.float32),
        pltpu.VMEM_SHARED((N_PAD,), jnp.float32),
    ],
)(_deg_body)


NBUF = 3


PH = 4
CPP = CHUNKS // PH


def _agg_body(y_hbm, src_hbm, dst_hbm, out_hbm, src_v, dst_v, buf_v, agg_sh, *sems):
    gsem = sems[:NBUF]
    ssem = sems[NBUF:]
    c = lax.axis_index("c")
    s = lax.axis_index("s")
    wid = c * NS + s
    pltpu.sync_copy(y_hbm.at[pl.ds(s * RPT, RPT)], agg_sh.at[pl.ds(s * RPT, RPT)])
    plsc.subcore_barrier()

    def gather(j, b):
        pltpu.async_copy(y_hbm.at[src_v.at[j]], buf_v.at[b], gsem[b])

    def gather_wait(j, b):
        pltpu.make_async_copy(y_hbm.at[src_v.at[j]], buf_v.at[b], gsem[b]).wait()

    def scat(j, b):
        pltpu.async_copy(buf_v.at[b], agg_sh.at[dst_v.at[j]], ssem[b], add=True)

    def scat_wait(j, b):
        pltpu.make_async_copy(buf_v.at[b], agg_sh.at[dst_v.at[j]], ssem[b]).wait()

    for p in range(PH):
        pltpu.sync_copy(src_hbm.at[wid, pl.ds(p * CPP, CPP)], src_v)
        pltpu.sync_copy(dst_hbm.at[wid, pl.ds(p * CPP, CPP)], dst_v)

        for b in range(NBUF):
            gather(b, b)

        def body(t, carry):
            j0 = t * NBUF
            for b in range(NBUF):
                gather_wait(j0 + b, b)
                scat(j0 + b, b)
            for b in range(NBUF):
                scat_wait(j0 + b, b)
                gather(j0 + NBUF + b, b)
            return carry

        lax.fori_loop(0, CPP // NBUF - 1, body, 0)
        j0 = CPP - NBUF
        for b in range(NBUF):
            gather_wait(j0 + b, b)
            scat(j0 + b, b)
        for b in range(NBUF):
            scat_wait(j0 + b, b)

    plsc.subcore_barrier()
    pltpu.sync_copy(
        agg_sh.at[pl.ds(s * RPT, RPT)], out_hbm.at[c, pl.ds(s * RPT, RPT)]
    )


_agg_call = functools.partial(
    pl.kernel,
    out_type=jax.ShapeDtypeStruct((NC, N_PAD, D), jnp.float32),
    mesh=_mesh(),
    scratch_types=[
        pltpu.VMEM((CPP, CHUNK), jnp.int32),
        pltpu.VMEM((CPP, CHUNK), jnp.int32),
        pltpu.VMEM((NBUF, CHUNK, D), jnp.float32),
        pltpu.VMEM_SHARED((N_PAD, D), jnp.float32),
    ]
    + [pltpu.SemaphoreType.DMA] * (2 * NBUF),
)(_agg_body)


def _scale_mm_body(hist_ref, x_ref, w_ref, dis_ref, y_ref):
    hist = hist_ref[...]
    dis = lax.rsqrt(hist[0] + hist[1] + 1.0)
    dis_ref[...] = dis
    xw = jnp.dot(x_ref[...], w_ref[...], preferred_element_type=jnp.float32)
    y_ref[...] = xw * dis


def _tc_scale_mm(hist3, xp, W1):
    return pl.pallas_call(
        _scale_mm_body,
        grid=(N_PAD // BLK,),
        in_specs=[
            pl.BlockSpec((NC, BLK, 1), lambda i: (0, i, 0)),
            pl.BlockSpec((BLK, D), lambda i: (i, 0)),
            pl.BlockSpec((D, D), lambda i: (0, 0)),
        ],
        out_specs=[
            pl.BlockSpec((BLK, 1), lambda i: (i, 0)),
            pl.BlockSpec((BLK, D), lambda i: (i, 0)),
        ],
        out_shape=[
            jax.ShapeDtypeStruct((N_PAD, 1), jnp.float32),
            jax.ShapeDtypeStruct((N_PAD, D), jnp.float32),
        ],
    )(hist3, xp, W1)


def _mid_body(agg_ref, y1_ref, dis_ref, b1_ref, w2_ref, y2_ref):
    a = agg_ref[...]
    dis = dis_ref[...]
    pre = dis * (a[0] + a[1] - y1_ref[...]) + b1_ref[...]
    h = jnp.maximum(pre, 0.0)
    hw = jnp.dot(h, w2_ref[...], preferred_element_type=jnp.float32)
    y2_ref[...] = hw * dis


def _tc_mid(agg1, y1, dis, b1, W2):
    return pl.pallas_call(
        _mid_body,
        grid=(N_PAD // BLK,),
        in_specs=[
            pl.BlockSpec((NC, BLK, D), lambda i: (0, i, 0)),
            pl.BlockSpec((BLK, D), lambda i: (i, 0)),
            pl.BlockSpec((BLK, 1), lambda i: (i, 0)),
            pl.BlockSpec((D,), lambda i: (0,)),
            pl.BlockSpec((D, D), lambda i: (0, 0)),
        ],
        out_specs=pl.BlockSpec((BLK, D), lambda i: (i, 0)),
        out_shape=jax.ShapeDtypeStruct((N_PAD, D), jnp.float32),
    )(agg1, y1, dis, b1, W2)


def _final_body(agg_ref, y2_ref, dis_ref, b2_ref, out_ref):
    a = agg_ref[...]
    z = dis_ref[...] * (a[0] + a[1] - y2_ref[...]) + b2_ref[...]
    m = jnp.max(z, axis=1, keepdims=True)
    lse = jnp.log(jnp.sum(jnp.exp(z - m), axis=1, keepdims=True))
    out_ref[...] = (z - m) - lse


def _tc_final(agg2, y2, dis, b2):
    return pl.pallas_call(
        _final_body,
        grid=(N_PAD // BLK,),
        in_specs=[
            pl.BlockSpec((NC, BLK, D), lambda i: (0, i, 0)),
            pl.BlockSpec((BLK, D), lambda i: (i, 0)),
            pl.BlockSpec((BLK, 1), lambda i: (i, 0)),
            pl.BlockSpec((D,), lambda i: (0,)),
        ],
        out_specs=pl.BlockSpec((BLK, D), lambda i: (i, 0)),
        out_shape=jax.ShapeDtypeStruct((N_PAD, D), jnp.float32),
    )(agg2, y2, dis, b2)


@jax.jit
def kernel(x, edge_index, W1, b1, W2, b2):
    src = edge_index[0].astype(jnp.int32)
    dst = edge_index[1].astype(jnp.int32)
    e = src.shape[0]
    pad = N + (jnp.arange(E_PAD - e, dtype=jnp.int32) % (N_PAD - N))
    srcp = jnp.concatenate([src, pad]).reshape(NW, CHUNKS, CHUNK)
    dstp = jnp.concatenate([dst, pad]).reshape(NW, CHUNKS, CHUNK)
    xp = jnp.pad(x, ((0, N_PAD - N), (0, 0)))

    hist = _deg_call(dstp)
    hist3 = hist.reshape(NC, N_PAD, 1)
    dis, y1 = _tc_scale_mm(hist3, xp, W1)
    agg1 = _agg_call(y1, srcp, dstp)
    y2 = _tc_mid(agg1, y1, dis, b1, W2)
    agg2 = _agg_call(y2, srcp, dstp)
    outp = _tc_final(agg2, y2, dis, b2)
    return outp[:N]
